# Initial kernel scaffold; baseline (speedup 1.0000x reference)
import jax
import jax.numpy as jnp
from jax.experimental import pallas as pl


def kernel(x, edge_index, edge_attr, u, batch, W1, b1, W2, b2):
    raise NotImplementedError("write your pallas kernel here")


# speedup vs baseline: 24.4513x; 24.4513x over previous
import functools

import jax
import jax.numpy as jnp
from jax import lax
from jax.experimental import pallas as pl
from jax.experimental.pallas import tpu as pltpu
from jax.experimental.pallas import tpu_sc as plsc

NC = 2
NS = 16
NW = NC * NS

CH = 1024

ZROWS = 6256


def _edge_scatter_body(n_nodes, n_edges, ecol0,
                       edge_flat, attr_flat, ones1, zeros1, out5,
                       idx_v, af, ones_v, zbuf, sem,
                       a0, a1, a2, a3, ct):
    c = lax.axis_index("c")
    s = lax.axis_index("s")
    wid = s * NC + c
    accs = (a0, a1, a2, a3, ct)

    pltpu.sync_copy(ones1, ones_v)
    pltpu.sync_copy(zeros1, zbuf)

    nb = s * ZROWS
    tail = n_nodes - (NS - 1) * ZROWS

    @pl.when(s < NS - 1)
    def _():
        for acc in accs:
            pltpu.sync_copy(zbuf, acc.at[pl.ds(nb, ZROWS)])

    @pl.when(s == NS - 1)
    def _():
        for acc in accs:
            pltpu.sync_copy(zbuf.at[pl.ds(0, tail)], acc.at[pl.ds(nb, tail)])

    plsc.subcore_barrier()

    STAGE = 3
    n_chunks = n_edges // CH
    n_iters = (n_chunks + NW - 1) // NW

    def step(t, carry):
        g = wid + t * NW

        @pl.when(g < n_chunks)
        def _():
            base = g * CH
            pltpu.sync_copy(edge_flat.at[1, pl.ds(base, CH)], idx_v)
            for f in range(4):
                pltpu.sync_copy(attr_flat.at[f, pl.ds(base, CH)],
                                af.at[f])
            if STAGE >= 3:
                descs = [pltpu.async_copy(af.at[f], accs[f].at[idx_v], sem,
                                          add=True)
                         for f in range(4)]
                descs.append(pltpu.async_copy(ones_v, ct.at[idx_v], sem,
                                              add=True))
                for d in descs:
                    d.wait()
        return carry

    if STAGE >= 2:
        lax.fori_loop(0, n_iters, step, 0)

    plsc.subcore_barrier()

    @pl.when(s < NS - 1)
    def _():
        for f, acc in enumerate(accs):
            pltpu.sync_copy(acc.at[pl.ds(nb, ZROWS)], zbuf)
            pltpu.sync_copy(zbuf, out5.at[c, f, pl.ds(nb, ZROWS)])

    @pl.when(s == NS - 1)
    def _():
        for f, acc in enumerate(accs):
            pltpu.sync_copy(acc.at[pl.ds(nb, tail)], zbuf.at[pl.ds(0, tail)])
            pltpu.sync_copy(zbuf.at[pl.ds(0, tail)],
                            out5.at[c, f, pl.ds(nb, tail)])


def _make_edge_scatter(n_nodes, n_edges):
    mesh = plsc.VectorSubcoreMesh(core_axis_name="c", subcore_axis_name="s")
    return functools.partial(
        pl.kernel,
        out_type=jax.ShapeDtypeStruct((NC, 5, n_nodes), jnp.float32),
        mesh=mesh,
        scratch_types=[
            pltpu.VMEM((CH,), jnp.int32),
            pltpu.VMEM((4, CH), jnp.float32),
            pltpu.VMEM((CH,), jnp.float32),
            pltpu.VMEM((ZROWS,), jnp.float32),
            pltpu.SemaphoreType.DMA,
            pltpu.VMEM_SHARED((n_nodes,), jnp.float32),
            pltpu.VMEM_SHARED((n_nodes,), jnp.float32),
            pltpu.VMEM_SHARED((n_nodes,), jnp.float32),
            pltpu.VMEM_SHARED((n_nodes,), jnp.float32),
            pltpu.VMEM_SHARED((n_nodes,), jnp.float32),
        ],
        compiler_params=pltpu.CompilerParams(use_tc_tiling_on_sc=False,
                                             needs_layout_passes=False),
    )(functools.partial(_edge_scatter_body, n_nodes, n_edges,
                        n_edges))


RB = 2000


def _combine_body(n_nodes, sums5, eut, b0, b1, b2, b3, c0, c1, eb):
    c = lax.axis_index("c")
    s = lax.axis_index("s")
    wid = s * NC + c
    n_chunks = n_nodes // RB
    n_iters = (n_chunks + NW - 1) // NW
    fbufs = (b0, b1, b2, b3)

    def step(t, carry):
        ch = wid + t * NW

        @pl.when(ch < n_chunks)
        def _():
            base = ch * RB
            pltpu.sync_copy(sums5.at[0, 4, pl.ds(base, RB)], c0)
            pltpu.sync_copy(sums5.at[1, 4, pl.ds(base, RB)], c1)
            for f in range(4):
                pltpu.sync_copy(sums5.at[0, f, pl.ds(base, RB)], fbufs[f])

            def inner_cnt(i, carry2):
                sl = pl.ds(i * 16, 16)
                cc = c0[sl] + c1[sl]
                c0[sl] = jnp.maximum(cc, jnp.full((16,), 1.0, jnp.float32))
                return carry2

            lax.fori_loop(0, RB // 16, inner_cnt, 0)

            for f in range(4):
                pltpu.sync_copy(sums5.at[1, f, pl.ds(base, RB)], c1)

                def inner(i, carry2, _f=f):
                    sl = pl.ds(i * 16, 16)
                    eb[sl] = (fbufs[_f][sl] + c1[sl]) / c0[sl]
                    return carry2

                lax.fori_loop(0, RB // 16, inner, 0)
                pltpu.sync_copy(eb, eut.at[ch, f, pl.ds(0, RB)])
        return carry

    lax.fori_loop(0, n_iters, step, 0)


def _make_combine(n_nodes):
    mesh = plsc.VectorSubcoreMesh(core_axis_name="c", subcore_axis_name="s")
    return functools.partial(
        pl.kernel,
        out_type=jax.ShapeDtypeStruct((n_nodes // RB, 8, RB), jnp.float32),
        mesh=mesh,
        scratch_types=[
            pltpu.VMEM((RB,), jnp.float32),
            pltpu.VMEM((RB,), jnp.float32),
            pltpu.VMEM((RB,), jnp.float32),
            pltpu.VMEM((RB,), jnp.float32),
            pltpu.VMEM((RB,), jnp.float32),
            pltpu.VMEM((RB,), jnp.float32),
            pltpu.VMEM((RB,), jnp.float32),
        ],
        compiler_params=pltpu.CompilerParams(use_tc_tiling_on_sc=False,
                                             needs_layout_passes=False),
    )(functools.partial(_combine_body, n_nodes))


def _graph_mlp_body(n_blocks, n_graphs,
                    xb, bb, eub, ub, w1b, b1b, w2b, b2b, ob, accx, acce):
    i = pl.program_id(0)
    bvec = bb[0, 0, :]
    mask = (bvec[:, None] ==
            lax.broadcasted_iota(jnp.int32, (1, n_graphs), 1)).astype(jnp.float32)

    xc = lax.dot_general(mask, xb[...], (((0,), (0,)), ((), ())),
                         preferred_element_type=jnp.float32,
                         precision=lax.Precision.HIGHEST)
    eu4 = eub[...][0, 0:4, :]
    eu8 = jnp.concatenate(
        [eu4, jnp.ones((1, RB), jnp.float32), jnp.zeros((3, RB), jnp.float32)],
        axis=0)
    ec = lax.dot_general(eu8, mask, (((1,), (0,)), ((), ())),
                         preferred_element_type=jnp.float32,
                         precision=lax.Precision.HIGHEST)

    @pl.when(i == 0)
    def _():
        accx[...] = xc
        acce[...] = ec

    @pl.when(i > 0)
    def _():
        accx[...] += xc
        acce[...] += ec

    @pl.when(i == n_blocks - 1)
    def _():
        cnts = jnp.maximum(acce[...][4:5, :], 1.0)
        xm = accx[...] / cnts.T
        em = (acce[...][0:4, :] / cnts).T
        cat = jnp.concatenate([ub[...], xm, em], axis=1)
        h1 = lax.dot_general(cat, w1b[...], (((1,), (0,)), ((), ())),
                             preferred_element_type=jnp.float32,
                             precision=lax.Precision.HIGHEST) + b1b[...]
        h1 = jnp.maximum(h1, 0.0)
        h2 = lax.dot_general(h1, w2b[...], (((1,), (0,)), ((), ())),
                             preferred_element_type=jnp.float32,
                             precision=lax.Precision.HIGHEST) + b2b[...]
        ob[...] = h2


def _graph_mlp(x, batch3, eut, u, w1, b1, w2, b2):
    n_nodes, fx = x.shape
    n_graphs, fu = u.shape
    in_ch = w1.shape[0]
    h1 = w1.shape[1]
    h2 = w2.shape[1]
    n_blocks = n_nodes // RB
    return pl.pallas_call(
        functools.partial(_graph_mlp_body, n_blocks, n_graphs),
        grid=(n_blocks,),
        in_specs=[
            pl.BlockSpec((RB, fx), lambda i: (i, 0)),
            pl.BlockSpec((1, 1, RB), lambda i: (i, 0, 0)),
            pl.BlockSpec((1, 8, RB), lambda i: (i, 0, 0)),
            pl.BlockSpec((n_graphs, fu), lambda i: (0, 0)),
            pl.BlockSpec((in_ch, h1), lambda i: (0, 0)),
            pl.BlockSpec((1, h1), lambda i: (0, 0)),
            pl.BlockSpec((h1, h2), lambda i: (0, 0)),
            pl.BlockSpec((1, h2), lambda i: (0, 0)),
        ],
        out_specs=pl.BlockSpec((n_graphs, h2), lambda i: (0, 0)),
        out_shape=jax.ShapeDtypeStruct((n_graphs, h2), jnp.float32),
        scratch_shapes=[
            pltpu.VMEM((n_graphs, fx), jnp.float32),
            pltpu.VMEM((8, n_graphs), jnp.float32),
        ],
    )(x, batch3, eut, u, w1, b1, w2, b2)


def kernel(x, edge_index, edge_attr, u, batch, W1, b1, W2, b2):
    n_nodes, fx = x.shape
    n_edges = edge_attr.shape[0]

    BISECT_FRESH_INPUTS = False
    if BISECT_FRESH_INPUTS:
        edge_flat = jnp.zeros((2, n_edges), jnp.int32) + edge_index[0, 0]
        attr_flat = jnp.zeros((4, n_edges), jnp.float32) + edge_attr[0, 0]
    else:
        edge_flat = edge_index
        attr_flat = edge_attr.T
    ones1 = jnp.ones((CH,), jnp.float32)
    zeros1 = jnp.zeros((ZROWS,), jnp.float32)

    sums5 = _make_edge_scatter(n_nodes, n_edges)(
        edge_flat, attr_flat, ones1, zeros1)
    eut = _make_combine(n_nodes)(sums5)

    batch3 = batch.reshape(n_nodes // RB, 1, RB)
    return _graph_mlp(x, batch3, eut, u,
                      W1, b1.reshape(1, -1), W2, b2.reshape(1, -1))


# speedup vs baseline: 49.0251x; 2.0050x over previous
import functools

import jax
import jax.numpy as jnp
from jax import lax
from jax.experimental import pallas as pl
from jax.experimental.pallas import tpu as pltpu
from jax.experimental.pallas import tpu_sc as plsc

NC = 2
NS = 16
NW = NC * NS

CH = 2048

ZROWS = 6256


def _edge_scatter_body(n_nodes, n_edges, ecol0,
                       edge_flat, attr_flat, ones1, zeros1, out5,
                       idx_v, af, ones_v, zbuf, sem_l, sem_s,
                       a0, a1, a2, a3, ct):
    c = lax.axis_index("c")
    s = lax.axis_index("s")
    wid = s * NC + c
    accs = (a0, a1, a2, a3, ct)

    pltpu.sync_copy(ones1, ones_v)
    pltpu.sync_copy(zeros1, zbuf)

    nb = s * ZROWS
    tail = n_nodes - (NS - 1) * ZROWS

    @pl.when(s < NS - 1)
    def _():
        for acc in accs:
            pltpu.sync_copy(zbuf, acc.at[pl.ds(nb, ZROWS)])

    @pl.when(s == NS - 1)
    def _():
        for acc in accs:
            pltpu.sync_copy(zbuf.at[pl.ds(0, tail)], acc.at[pl.ds(nb, tail)])

    plsc.subcore_barrier()

    n_chunks = n_edges // CH
    n_iters = (n_chunks + NW - 1) // NW

    def fire_loads(g, p):
        base = g * CH
        pltpu.async_copy(edge_flat.at[1, pl.ds(base, CH)], idx_v.at[p], sem_l)
        for f in range(4):
            pltpu.async_copy(attr_flat.at[f, pl.ds(base, CH)], af.at[p, f],
                             sem_l)

    def wait_loads(g, p):
        base = g * CH
        pltpu.make_async_copy(edge_flat.at[1, pl.ds(base, CH)], idx_v.at[p],
                              sem_l).wait()
        for f in range(4):
            pltpu.make_async_copy(attr_flat.at[f, pl.ds(base, CH)],
                                  af.at[p, f], sem_l).wait()

    def fire_scatters(p):
        for f in range(4):
            pltpu.async_copy(af.at[p, f], accs[f].at[idx_v.at[p]], sem_s,
                             add=True)
        pltpu.async_copy(ones_v, ct.at[idx_v.at[p]], sem_s, add=True)

    def drain_scatters(p):
        for f in range(4):
            pltpu.make_async_copy(af.at[p, f], accs[f].at[idx_v.at[p]],
                                  sem_s).wait()
        pltpu.make_async_copy(ones_v, ct.at[idx_v.at[p]], sem_s).wait()

    fire_loads(wid, 0)

    def step2(t2, carry):
        for p in range(2):
            t = t2 * 2 + p
            g = wid + t * NW
            g_next = g + NW

            @pl.when((t > 0) & (g - NW < n_chunks))
            def _():
                drain_scatters(1 - p)

            @pl.when(g_next < n_chunks)
            def _():
                fire_loads(g_next, 1 - p)

            @pl.when(g < n_chunks)
            def _():
                wait_loads(g, p)
                fire_scatters(p)
        return carry

    assert n_iters % 2 == 0
    lax.fori_loop(0, n_iters // 2, step2, 0)

    g_last = wid + (n_iters - 1) * NW

    @pl.when(g_last < n_chunks)
    def _():
        drain_scatters((n_iters - 1) % 2)

    plsc.subcore_barrier()

    @pl.when(s < NS - 1)
    def _():
        for f, acc in enumerate(accs):
            pltpu.sync_copy(acc.at[pl.ds(nb, ZROWS)], zbuf)
            pltpu.sync_copy(zbuf, out5.at[c, f, pl.ds(nb, ZROWS)])

    @pl.when(s == NS - 1)
    def _():
        for f, acc in enumerate(accs):
            pltpu.sync_copy(acc.at[pl.ds(nb, tail)], zbuf.at[pl.ds(0, tail)])
            pltpu.sync_copy(zbuf.at[pl.ds(0, tail)],
                            out5.at[c, f, pl.ds(nb, tail)])


def _make_edge_scatter(n_nodes, n_edges):
    mesh = plsc.VectorSubcoreMesh(core_axis_name="c", subcore_axis_name="s")
    return functools.partial(
        pl.kernel,
        out_type=jax.ShapeDtypeStruct((NC, 5, n_nodes), jnp.float32),
        mesh=mesh,
        scratch_types=[
            pltpu.VMEM((2, CH), jnp.int32),
            pltpu.VMEM((2, 4, CH), jnp.float32),
            pltpu.VMEM((CH,), jnp.float32),
            pltpu.VMEM((ZROWS,), jnp.float32),
            pltpu.SemaphoreType.DMA,
            pltpu.SemaphoreType.DMA,
            pltpu.VMEM_SHARED((n_nodes,), jnp.float32),
            pltpu.VMEM_SHARED((n_nodes,), jnp.float32),
            pltpu.VMEM_SHARED((n_nodes,), jnp.float32),
            pltpu.VMEM_SHARED((n_nodes,), jnp.float32),
            pltpu.VMEM_SHARED((n_nodes,), jnp.float32),
        ],
        compiler_params=pltpu.CompilerParams(use_tc_tiling_on_sc=False,
                                             needs_layout_passes=False),
    )(functools.partial(_edge_scatter_body, n_nodes, n_edges,
                        n_edges))


RB = 2000


def _combine_body(n_nodes, sums5, eut, b0, b1, b2, b3, c0, c1, eb):
    c = lax.axis_index("c")
    s = lax.axis_index("s")
    wid = s * NC + c
    n_chunks = n_nodes // RB
    n_iters = (n_chunks + NW - 1) // NW
    fbufs = (b0, b1, b2, b3)

    def step(t, carry):
        ch = wid + t * NW

        @pl.when(ch < n_chunks)
        def _():
            base = ch * RB
            pltpu.sync_copy(sums5.at[0, 4, pl.ds(base, RB)], c0)
            pltpu.sync_copy(sums5.at[1, 4, pl.ds(base, RB)], c1)
            for f in range(4):
                pltpu.sync_copy(sums5.at[0, f, pl.ds(base, RB)], fbufs[f])

            def inner_cnt(i, carry2):
                sl = pl.ds(i * 16, 16)
                cc = c0[sl] + c1[sl]
                c0[sl] = jnp.maximum(cc, jnp.full((16,), 1.0, jnp.float32))
                return carry2

            lax.fori_loop(0, RB // 16, inner_cnt, 0)

            for f in range(4):
                pltpu.sync_copy(sums5.at[1, f, pl.ds(base, RB)], c1)

                def inner(i, carry2, _f=f):
                    sl = pl.ds(i * 16, 16)
                    eb[sl] = (fbufs[_f][sl] + c1[sl]) / c0[sl]
                    return carry2

                lax.fori_loop(0, RB // 16, inner, 0)
                pltpu.sync_copy(eb, eut.at[ch, f, pl.ds(0, RB)])
        return carry

    lax.fori_loop(0, n_iters, step, 0)


def _make_combine(n_nodes):
    mesh = plsc.VectorSubcoreMesh(core_axis_name="c", subcore_axis_name="s")
    return functools.partial(
        pl.kernel,
        out_type=jax.ShapeDtypeStruct((n_nodes // RB, 8, RB), jnp.float32),
        mesh=mesh,
        scratch_types=[
            pltpu.VMEM((RB,), jnp.float32),
            pltpu.VMEM((RB,), jnp.float32),
            pltpu.VMEM((RB,), jnp.float32),
            pltpu.VMEM((RB,), jnp.float32),
            pltpu.VMEM((RB,), jnp.float32),
            pltpu.VMEM((RB,), jnp.float32),
            pltpu.VMEM((RB,), jnp.float32),
        ],
        compiler_params=pltpu.CompilerParams(use_tc_tiling_on_sc=False,
                                             needs_layout_passes=False),
    )(functools.partial(_combine_body, n_nodes))


def _graph_mlp_body(n_blocks, n_graphs,
                    xb, bb, eub, ub, w1b, b1b, w2b, b2b, ob, accx, acce):
    i = pl.program_id(0)
    bvec = bb[0, 0, :]
    mask = (bvec[:, None] ==
            lax.broadcasted_iota(jnp.int32, (1, n_graphs), 1)).astype(jnp.float32)

    xc = lax.dot_general(mask, xb[...], (((0,), (0,)), ((), ())),
                         preferred_element_type=jnp.float32,
                         precision=lax.Precision.HIGHEST)
    eu4 = eub[...][0, 0:4, :]
    eu8 = jnp.concatenate(
        [eu4, jnp.ones((1, RB), jnp.float32), jnp.zeros((3, RB), jnp.float32)],
        axis=0)
    ec = lax.dot_general(eu8, mask, (((1,), (0,)), ((), ())),
                         preferred_element_type=jnp.float32,
                         precision=lax.Precision.HIGHEST)

    @pl.when(i == 0)
    def _():
        accx[...] = xc
        acce[...] = ec

    @pl.when(i > 0)
    def _():
        accx[...] += xc
        acce[...] += ec

    @pl.when(i == n_blocks - 1)
    def _():
        cnts = jnp.maximum(acce[...][4:5, :], 1.0)
        xm = accx[...] / cnts.T
        em = (acce[...][0:4, :] / cnts).T
        cat = jnp.concatenate([ub[...], xm, em], axis=1)
        h1 = lax.dot_general(cat, w1b[...], (((1,), (0,)), ((), ())),
                             preferred_element_type=jnp.float32,
                             precision=lax.Precision.HIGHEST) + b1b[...]
        h1 = jnp.maximum(h1, 0.0)
        h2 = lax.dot_general(h1, w2b[...], (((1,), (0,)), ((), ())),
                             preferred_element_type=jnp.float32,
                             precision=lax.Precision.HIGHEST) + b2b[...]
        ob[...] = h2


def _graph_mlp(x, batch3, eut, u, w1, b1, w2, b2):
    n_nodes, fx = x.shape
    n_graphs, fu = u.shape
    in_ch = w1.shape[0]
    h1 = w1.shape[1]
    h2 = w2.shape[1]
    n_blocks = n_nodes // RB
    return pl.pallas_call(
        functools.partial(_graph_mlp_body, n_blocks, n_graphs),
        grid=(n_blocks,),
        in_specs=[
            pl.BlockSpec((RB, fx), lambda i: (i, 0)),
            pl.BlockSpec((1, 1, RB), lambda i: (i, 0, 0)),
            pl.BlockSpec((1, 8, RB), lambda i: (i, 0, 0)),
            pl.BlockSpec((n_graphs, fu), lambda i: (0, 0)),
            pl.BlockSpec((in_ch, h1), lambda i: (0, 0)),
            pl.BlockSpec((1, h1), lambda i: (0, 0)),
            pl.BlockSpec((h1, h2), lambda i: (0, 0)),
            pl.BlockSpec((1, h2), lambda i: (0, 0)),
        ],
        out_specs=pl.BlockSpec((n_graphs, h2), lambda i: (0, 0)),
        out_shape=jax.ShapeDtypeStruct((n_graphs, h2), jnp.float32),
        scratch_shapes=[
            pltpu.VMEM((n_graphs, fx), jnp.float32),
            pltpu.VMEM((8, n_graphs), jnp.float32),
        ],
    )(x, batch3, eut, u, w1, b1, w2, b2)


def kernel(x, edge_index, edge_attr, u, batch, W1, b1, W2, b2):
    n_nodes, fx = x.shape
    n_edges = edge_attr.shape[0]

    edge_flat = edge_index
    attr_flat = edge_attr.T
    ones1 = jnp.ones((CH,), jnp.float32)
    zeros1 = jnp.zeros((ZROWS,), jnp.float32)

    sums5 = _make_edge_scatter(n_nodes, n_edges)(
        edge_flat, attr_flat, ones1, zeros1)
    eut = _make_combine(n_nodes)(sums5)

    batch3 = batch.reshape(n_nodes // RB, 1, RB)
    return _graph_mlp(x, batch3, eut, u,
                      W1, b1.reshape(1, -1), W2, b2.reshape(1, -1))


# speedup vs baseline: 49.5656x; 1.0110x over previous
import functools

import jax
import jax.numpy as jnp
from jax import lax
from jax.experimental import pallas as pl
from jax.experimental.pallas import tpu as pltpu
from jax.experimental.pallas import tpu_sc as plsc

NC = 2
NS = 16
NW = NC * NS

CH = 4096

ZROWS = 6256


def _edge_scatter_body(n_nodes, n_edges, ecol0,
                       edge_flat, attr_flat, ones1, zeros1, out5,
                       idx_v, af, ones_v, zbuf, sem_l, sem_s,
                       a0, a1, a2, a3, ct):
    c = lax.axis_index("c")
    s = lax.axis_index("s")
    wid = s * NC + c
    accs = (a0, a1, a2, a3, ct)

    pltpu.sync_copy(ones1, ones_v)
    pltpu.sync_copy(zeros1, zbuf)

    nb = s * ZROWS
    tail = n_nodes - (NS - 1) * ZROWS

    @pl.when(s < NS - 1)
    def _():
        for acc in accs:
            pltpu.sync_copy(zbuf, acc.at[pl.ds(nb, ZROWS)])

    @pl.when(s == NS - 1)
    def _():
        for acc in accs:
            pltpu.sync_copy(zbuf.at[pl.ds(0, tail)], acc.at[pl.ds(nb, tail)])

    plsc.subcore_barrier()

    n_chunks = n_edges // CH
    n_iters = (n_chunks + NW - 1) // NW

    def fire_loads(g, p):
        base = g * CH
        pltpu.async_copy(edge_flat.at[1, pl.ds(base, CH)], idx_v.at[p], sem_l)
        for f in range(4):
            pltpu.async_copy(attr_flat.at[f, pl.ds(base, CH)], af.at[p, f],
                             sem_l)

    def wait_loads(g, p):
        base = g * CH
        pltpu.make_async_copy(edge_flat.at[1, pl.ds(base, CH)], idx_v.at[p],
                              sem_l).wait()
        for f in range(4):
            pltpu.make_async_copy(attr_flat.at[f, pl.ds(base, CH)],
                                  af.at[p, f], sem_l).wait()

    def fire_scatters(p):
        for f in range(4):
            pltpu.async_copy(af.at[p, f], accs[f].at[idx_v.at[p]], sem_s,
                             add=True)
        pltpu.async_copy(ones_v, ct.at[idx_v.at[p]], sem_s, add=True)

    def drain_scatters(p):
        for f in range(4):
            pltpu.make_async_copy(af.at[p, f], accs[f].at[idx_v.at[p]],
                                  sem_s).wait()
        pltpu.make_async_copy(ones_v, ct.at[idx_v.at[p]], sem_s).wait()

    fire_loads(wid, 0)

    def step2(t2, carry):
        for p in range(2):
            t = t2 * 2 + p
            g = wid + t * NW
            g_next = g + NW

            @pl.when((t > 0) & (g - NW < n_chunks))
            def _():
                drain_scatters(1 - p)

            @pl.when(g_next < n_chunks)
            def _():
                fire_loads(g_next, 1 - p)

            @pl.when(g < n_chunks)
            def _():
                wait_loads(g, p)
                fire_scatters(p)
        return carry

    n_eff = 2 * ((n_iters + 1) // 2)
    lax.fori_loop(0, n_eff // 2, step2, 0)

    g_last = wid + (n_eff - 1) * NW

    @pl.when(g_last < n_chunks)
    def _():
        drain_scatters((n_eff - 1) % 2)

    plsc.subcore_barrier()

    @pl.when(s < NS - 1)
    def _():
        for f, acc in enumerate(accs):
            pltpu.sync_copy(acc.at[pl.ds(nb, ZROWS)], zbuf)
            pltpu.sync_copy(zbuf, out5.at[c, f, pl.ds(nb, ZROWS)])

    @pl.when(s == NS - 1)
    def _():
        for f, acc in enumerate(accs):
            pltpu.sync_copy(acc.at[pl.ds(nb, tail)], zbuf.at[pl.ds(0, tail)])
            pltpu.sync_copy(zbuf.at[pl.ds(0, tail)],
                            out5.at[c, f, pl.ds(nb, tail)])


def _make_edge_scatter(n_nodes, n_edges):
    mesh = plsc.VectorSubcoreMesh(core_axis_name="c", subcore_axis_name="s")
    return functools.partial(
        pl.kernel,
        out_type=jax.ShapeDtypeStruct((NC, 5, n_nodes), jnp.float32),
        mesh=mesh,
        scratch_types=[
            pltpu.VMEM((2, CH), jnp.int32),
            pltpu.VMEM((2, 4, CH), jnp.float32),
            pltpu.VMEM((CH,), jnp.float32),
            pltpu.VMEM((ZROWS,), jnp.float32),
            pltpu.SemaphoreType.DMA,
            pltpu.SemaphoreType.DMA,
            pltpu.VMEM_SHARED((n_nodes,), jnp.float32),
            pltpu.VMEM_SHARED((n_nodes,), jnp.float32),
            pltpu.VMEM_SHARED((n_nodes,), jnp.float32),
            pltpu.VMEM_SHARED((n_nodes,), jnp.float32),
            pltpu.VMEM_SHARED((n_nodes,), jnp.float32),
        ],
        compiler_params=pltpu.CompilerParams(use_tc_tiling_on_sc=False,
                                             needs_layout_passes=False),
    )(functools.partial(_edge_scatter_body, n_nodes, n_edges,
                        n_edges))


RB = 2000


def _combine_body(n_nodes, sums5, eut, b0, b1, b2, b3, c0, c1, eb):
    c = lax.axis_index("c")
    s = lax.axis_index("s")
    wid = s * NC + c
    n_chunks = n_nodes // RB
    n_iters = (n_chunks + NW - 1) // NW
    fbufs = (b0, b1, b2, b3)

    def step(t, carry):
        ch = wid + t * NW

        @pl.when(ch < n_chunks)
        def _():
            base = ch * RB
            pltpu.sync_copy(sums5.at[0, 4, pl.ds(base, RB)], c0)
            pltpu.sync_copy(sums5.at[1, 4, pl.ds(base, RB)], c1)
            for f in range(4):
                pltpu.sync_copy(sums5.at[0, f, pl.ds(base, RB)], fbufs[f])

            def inner_cnt(i, carry2):
                sl = pl.ds(i * 16, 16)
                cc = c0[sl] + c1[sl]
                c0[sl] = jnp.maximum(cc, jnp.full((16,), 1.0, jnp.float32))
                return carry2

            lax.fori_loop(0, RB // 16, inner_cnt, 0)

            for f in range(4):
                pltpu.sync_copy(sums5.at[1, f, pl.ds(base, RB)], c1)

                def inner(i, carry2, _f=f):
                    sl = pl.ds(i * 16, 16)
                    eb[sl] = (fbufs[_f][sl] + c1[sl]) / c0[sl]
                    return carry2

                lax.fori_loop(0, RB // 16, inner, 0)
                pltpu.sync_copy(eb, eut.at[ch, f, pl.ds(0, RB)])
        return carry

    lax.fori_loop(0, n_iters, step, 0)


def _make_combine(n_nodes):
    mesh = plsc.VectorSubcoreMesh(core_axis_name="c", subcore_axis_name="s")
    return functools.partial(
        pl.kernel,
        out_type=jax.ShapeDtypeStruct((n_nodes // RB, 8, RB), jnp.float32),
        mesh=mesh,
        scratch_types=[
            pltpu.VMEM((RB,), jnp.float32),
            pltpu.VMEM((RB,), jnp.float32),
            pltpu.VMEM((RB,), jnp.float32),
            pltpu.VMEM((RB,), jnp.float32),
            pltpu.VMEM((RB,), jnp.float32),
            pltpu.VMEM((RB,), jnp.float32),
            pltpu.VMEM((RB,), jnp.float32),
        ],
        compiler_params=pltpu.CompilerParams(use_tc_tiling_on_sc=False,
                                             needs_layout_passes=False),
    )(functools.partial(_combine_body, n_nodes))


def _x_pass_body(n_blocks, n_graphs, xb, bb, xout, cout, accx, accc):
    i = pl.program_id(0)
    bvec = bb[0, 0, :]
    mask = (bvec[:, None] ==
            lax.broadcasted_iota(jnp.int32, (1, n_graphs), 1)).astype(jnp.float32)

    xc = lax.dot_general(mask, xb[...], (((0,), (0,)), ((), ())),
                         preferred_element_type=jnp.float32,
                         precision=lax.Precision.HIGHEST)
    cc = jnp.sum(mask, axis=0, keepdims=True)

    @pl.when(i == 0)
    def _():
        accx[...] = xc
        accc[...] = cc

    @pl.when(i > 0)
    def _():
        accx[...] += xc
        accc[...] += cc

    @pl.when(i == n_blocks - 1)
    def _():
        xout[...] = accx[...]
        cout[...] = accc[...]


def _x_pass(x, batch3):
    n_nodes, fx = x.shape
    n_graphs = 64
    n_blocks = n_nodes // RB
    return pl.pallas_call(
        functools.partial(_x_pass_body, n_blocks, n_graphs),
        grid=(n_blocks,),
        in_specs=[
            pl.BlockSpec((RB, fx), lambda i: (i, 0)),
            pl.BlockSpec((1, 1, RB), lambda i: (i, 0, 0)),
        ],
        out_specs=[
            pl.BlockSpec((n_graphs, fx), lambda i: (0, 0)),
            pl.BlockSpec((1, n_graphs), lambda i: (0, 0)),
        ],
        out_shape=[
            jax.ShapeDtypeStruct((n_graphs, fx), jnp.float32),
            jax.ShapeDtypeStruct((1, n_graphs), jnp.float32),
        ],
        scratch_shapes=[
            pltpu.VMEM((n_graphs, fx), jnp.float32),
            pltpu.VMEM((1, n_graphs), jnp.float32),
        ],
    )(x, batch3)


def _e_mlp_body(n_blocks, n_graphs,
                eub, bb, xsb, csb, ub, w1b, b1b, w2b, b2b, ob, acce):
    i = pl.program_id(0)
    bvec = bb[0, 0, :]
    mask = (bvec[:, None] ==
            lax.broadcasted_iota(jnp.int32, (1, n_graphs), 1)).astype(jnp.float32)

    eu4 = eub[...][0, 0:4, :]
    eu8 = jnp.concatenate(
        [eu4, jnp.zeros((4, RB), jnp.float32)], axis=0)
    ec = lax.dot_general(eu8, mask, (((1,), (0,)), ((), ())),
                         preferred_element_type=jnp.float32,
                         precision=lax.Precision.HIGHEST)

    @pl.when(i == 0)
    def _():
        acce[...] = ec

    @pl.when(i > 0)
    def _():
        acce[...] += ec

    @pl.when(i == n_blocks - 1)
    def _():
        cnts = jnp.maximum(csb[...], 1.0)
        xm = xsb[...] / cnts.T
        em = (acce[...][0:4, :] / cnts).T
        cat = jnp.concatenate([ub[...], xm, em], axis=1)
        h1 = lax.dot_general(cat, w1b[...], (((1,), (0,)), ((), ())),
                             preferred_element_type=jnp.float32,
                             precision=lax.Precision.HIGHEST) + b1b[...]
        h1 = jnp.maximum(h1, 0.0)
        h2 = lax.dot_general(h1, w2b[...], (((1,), (0,)), ((), ())),
                             preferred_element_type=jnp.float32,
                             precision=lax.Precision.HIGHEST) + b2b[...]
        ob[...] = h2


def _e_mlp(eut, batch3, xsum, csum, u, w1, b1, w2, b2):
    n_graphs, fu = u.shape
    fx = xsum.shape[1]
    in_ch = w1.shape[0]
    h1 = w1.shape[1]
    h2 = w2.shape[1]
    n_blocks = eut.shape[0]
    return pl.pallas_call(
        functools.partial(_e_mlp_body, n_blocks, n_graphs),
        grid=(n_blocks,),
        in_specs=[
            pl.BlockSpec((1, 8, RB), lambda i: (i, 0, 0)),
            pl.BlockSpec((1, 1, RB), lambda i: (i, 0, 0)),
            pl.BlockSpec((n_graphs, fx), lambda i: (0, 0)),
            pl.BlockSpec((1, n_graphs), lambda i: (0, 0)),
            pl.BlockSpec((n_graphs, fu), lambda i: (0, 0)),
            pl.BlockSpec((in_ch, h1), lambda i: (0, 0)),
            pl.BlockSpec((1, h1), lambda i: (0, 0)),
            pl.BlockSpec((h1, h2), lambda i: (0, 0)),
            pl.BlockSpec((1, h2), lambda i: (0, 0)),
        ],
        out_specs=pl.BlockSpec((n_graphs, h2), lambda i: (0, 0)),
        out_shape=jax.ShapeDtypeStruct((n_graphs, h2), jnp.float32),
        scratch_shapes=[
            pltpu.VMEM((8, n_graphs), jnp.float32),
        ],
    )(eut, batch3, xsum, csum, u, w1, b1, w2, b2)


def kernel(x, edge_index, edge_attr, u, batch, W1, b1, W2, b2):
    n_nodes, fx = x.shape
    n_edges = edge_attr.shape[0]

    edge_flat = edge_index
    attr_flat = edge_attr.T
    ones1 = jnp.ones((CH,), jnp.float32)
    zeros1 = jnp.zeros((ZROWS,), jnp.float32)

    batch3 = batch.reshape(n_nodes // RB, 1, RB)
    xsum, csum = _x_pass(x, batch3)

    sums5 = _make_edge_scatter(n_nodes, n_edges)(
        edge_flat, attr_flat, ones1, zeros1)
    eut = _make_combine(n_nodes)(sums5)

    return _e_mlp(eut, batch3, xsum, csum, u,
                  W1, b1.reshape(1, -1), W2, b2.reshape(1, -1))
